# Initial kernel scaffold; baseline (speedup 1.0000x reference)
import jax
import jax.numpy as jnp
from jax.experimental import pallas as pl


def kernel(x, Wl1, bl1, Wr1, Wl2, bl2, Wr2, Wa, ba, edge_index):
    raise NotImplementedError("write your pallas kernel here")


# speedup vs baseline: 4.3892x; 4.3892x over previous
import jax
import jax.numpy as jnp
from jax import lax
from jax.experimental import pallas as pl
from jax.experimental.pallas import tpu as pltpu
from jax.experimental.pallas import tpu_sc as plsc

NC = 2
NS = 16
NW = NC * NS
C = 128
W = 128


def _tc1_body(x_ref, wl_ref, wr_ref, t_ref, z_ref):
    xb = x_ref[...]
    p = jnp.dot(xb, wl_ref[...], preferred_element_type=jnp.float32)
    col = lax.broadcasted_iota(jnp.int32, p.shape, 1)
    t_ref[...] = p + jnp.where(col == 64, 1.0, 0.0)
    z_ref[...] = jnp.dot(xb, wr_ref[...], preferred_element_type=jnp.float32)


def _tc1(x, wl, wr, blk, n_acc):
    n, din = x.shape
    dz = wr.shape[1]
    return pl.pallas_call(
        _tc1_body,
        grid=(n // blk,),
        in_specs=[
            pl.BlockSpec((blk, din), lambda i: (i, 0)),
            pl.BlockSpec((din, W), lambda i: (0, 0)),
            pl.BlockSpec((din, dz), lambda i: (0, 0)),
        ],
        out_specs=[
            pl.BlockSpec((blk, W), lambda i: (i, 0)),
            pl.BlockSpec((blk, dz), lambda i: (i, 0)),
        ],
        out_shape=[
            jax.ShapeDtypeStruct((n_acc, W), jnp.float32),
            jax.ShapeDtypeStruct((n, dz), jnp.float32),
        ],
    )(x, wl, wr)


def _make_sc_pass(n_acc, cpw):
    rpt = n_acc // NS
    zc = rpt // C
    mesh = plsc.VectorSubcoreMesh(core_axis_name="c", subcore_axis_name="s")
    out_type = jax.ShapeDtypeStruct((NC, n_acc, W), jnp.float32)
    scratch = [
        pltpu.VMEM((C,), jnp.int32),
        pltpu.VMEM((C,), jnp.int32),
        pltpu.VMEM((C, W), jnp.float32),
        pltpu.VMEM_SHARED((n_acc, W), jnp.float32),
        pltpu.SemaphoreType.DMA,
    ]

    def body(tbl_hbm, src_hbm, dst_hbm, seg_out, src_v, dst_v, rows_v,
             acc_sh, sem):
        c = lax.axis_index("c")
        s = lax.axis_index("s")
        wid = s * NC + c

        zero16 = jnp.zeros((16,), jnp.float32)

        def fill(i, _):
            for j in range(W // 16):
                rows_v[i, pl.ds(j * 16, 16)] = zero16
            return 0

        lax.fori_loop(0, C, fill, 0)

        for k in range(zc):
            pltpu.sync_copy(rows_v, acc_sh.at[pl.ds(s * rpt + k * C, C)])

        plsc.subcore_barrier()

        def step(t, _):
            ebase = (wid * cpw + t) * C
            pltpu.sync_copy(src_hbm.at[pl.ds(ebase, C)], src_v)
            pltpu.sync_copy(dst_hbm.at[pl.ds(ebase, C)], dst_v)
            pltpu.async_copy(tbl_hbm.at[src_v], rows_v, sem).wait()
            pltpu.sync_copy(rows_v, acc_sh.at[dst_v], add=True)
            return 0

        lax.fori_loop(0, cpw, step, 0)

        plsc.subcore_barrier()

        ob = s * rpt
        pltpu.sync_copy(acc_sh.at[pl.ds(ob, rpt)], seg_out.at[c, pl.ds(ob, rpt)])

    return pl.kernel(body, out_type=out_type, mesh=mesh,
                     scratch_types=scratch)


def _tc2_body(s_ref, z_ref, bl_ref, wl_ref, wr_ref, t_ref, z2_ref):
    sa = s_ref[0] + s_ref[1]
    cnt = jnp.sum(sa[:, 64:], axis=1, keepdims=True)
    r = 1.0 / jnp.maximum(cnt, 1.0)
    h = jnp.maximum(sa[:, :64] * r + bl_ref[...] + z_ref[...], 0.0)
    t_ref[...] = jnp.dot(h, wl_ref[...], preferred_element_type=jnp.float32)
    z2_ref[...] = jnp.dot(h, wr_ref[...], preferred_element_type=jnp.float32)


def _tc2(seg, z1, bl, wl, wr, blk, n_acc):
    n, d1 = z1.shape
    dz = wr.shape[1]
    return pl.pallas_call(
        _tc2_body,
        grid=(n // blk,),
        in_specs=[
            pl.BlockSpec((NC, blk, W), lambda i: (0, i, 0)),
            pl.BlockSpec((blk, d1), lambda i: (i, 0)),
            pl.BlockSpec((1, d1), lambda i: (0, 0)),
            pl.BlockSpec((d1, W), lambda i: (0, 0)),
            pl.BlockSpec((d1, dz), lambda i: (0, 0)),
        ],
        out_specs=[
            pl.BlockSpec((blk, W), lambda i: (i, 0)),
            pl.BlockSpec((blk, dz), lambda i: (i, 0)),
        ],
        out_shape=[
            jax.ShapeDtypeStruct((n_acc, W), jnp.float32),
            jax.ShapeDtypeStruct((n, dz), jnp.float32),
        ],
    )(seg, z1, bl, wl, wr)


def _tc3_body(sa_ref, sb_ref, z_ref, bl_ref, wa_ref, ba_ref, out_ref):
    sa = sa_ref[0] + sa_ref[1]
    cnt = jnp.sum(sa[:, 64:], axis=1, keepdims=True)
    r = 1.0 / jnp.maximum(cnt, 1.0)
    sb = sb_ref[0] + sb_ref[1]
    d2 = z_ref.shape[1]
    h = jnp.maximum(sb[:, :d2] * r + bl_ref[...] + z_ref[...], 0.0)
    ang = jnp.sum(h * wa_ref[...], axis=1, keepdims=True) + ba_ref[...]
    cth = jnp.cos(ang)
    sth = jnp.sin(ang)
    col = lax.broadcasted_iota(jnp.int32, out_ref.shape, 1)
    out_ref[...] = jnp.where((col == 0) | (col == 3), cth,
                             jnp.where(col == 1, -sth, sth))


def _tc3(segA, segB, z2, bl, wa, ba, blk):
    n, d2 = z2.shape
    return pl.pallas_call(
        _tc3_body,
        grid=(n // blk,),
        in_specs=[
            pl.BlockSpec((NC, blk, W), lambda i: (0, i, 0)),
            pl.BlockSpec((NC, blk, W), lambda i: (0, i, 0)),
            pl.BlockSpec((blk, d2), lambda i: (i, 0)),
            pl.BlockSpec((1, d2), lambda i: (0, 0)),
            pl.BlockSpec((1, d2), lambda i: (0, 0)),
            pl.BlockSpec((1, 1), lambda i: (0, 0)),
        ],
        out_specs=pl.BlockSpec((blk, 4), lambda i: (i, 0)),
        out_shape=jax.ShapeDtypeStruct((n, 4), jnp.float32),
    )(segA, segB, z2, bl, wa, ba)


def kernel(x, Wl1, bl1, Wr1, Wl2, bl2, Wr2, Wa, ba, edge_index):
    n, din = x.shape
    E = edge_index.shape[1]
    d1 = Wl1.shape[0]
    d2 = Wl2.shape[0]

    epc = NW * C
    EP = ((E + epc - 1) // epc) * epc
    cpw = EP // epc
    src = edge_index[0]
    dst = edge_index[1]
    if EP != E:
        pad = EP - E
        src = jnp.concatenate([src, jnp.zeros((pad,), jnp.int32)])
        dst = jnp.concatenate([dst, jnp.full((pad,), n, jnp.int32)])

    n_acc = ((n + 1 + NS * C - 1) // (NS * C)) * (NS * C)
    blk = 1000 if n % 1000 == 0 else n

    wl1p = jnp.concatenate([Wl1.T, jnp.zeros((din, W - d1), jnp.float32)], 1)
    wl2p = jnp.concatenate([Wl2.T, jnp.zeros((d1, W - d2), jnp.float32)], 1)

    tblA, z1 = _tc1(x, wl1p, Wr1.T, blk, n_acc)
    sc = _make_sc_pass(n_acc, cpw)
    segA = sc(tblA, src, dst)
    tblB, z2 = _tc2(segA, z1, bl1.reshape(1, d1), wl2p, Wr2.T, blk, n_acc)
    segB = sc(tblB, src, dst)
    out = _tc3(segA, segB, z2, bl2.reshape(1, d2), Wa.reshape(1, d2),
               ba.reshape(1, 1), blk)
    return out.reshape(n, 2, 2)
